# baseline (speedup 1.0000x reference)
import functools
import jax
import jax.numpy as jnp
from jax import lax
from jax.experimental import pallas as pl
from jax.experimental.pallas import tpu as pltpu
from jax.experimental.pallas import tpu_sc as plsc

X_ID = 343
NBUF = 8
LANES = 16


def _lane_extract(vec, lane):
    lanes = lax.iota(jnp.int32, LANES)
    return jnp.max(jnp.where(lanes == lane, vec, -1))


def _make_sc_kernel(b, t, d, n_ctx, dtype):
    info = plsc.get_sparse_core_info()
    nc, ns = info.num_cores, info.num_subcores
    nw = nc * ns
    s_per_w = b // nw
    n_chunks = s_per_w // LANES
    mesh = plsc.VectorSubcoreMesh(core_axis_name="c", subcore_axis_name="s")

    @functools.partial(
        pl.kernel,
        out_type=jax.ShapeDtypeStruct((b, t, d), dtype),
        scratch_types=[
            pltpu.VMEM((2, t, d), dtype),
            pltpu.VMEM((s_per_w,), jnp.int32),
            pltpu.VMEM((LANES,), jnp.int32),
            pltpu.SemaphoreType.DMA((NBUF,)),
        ],
        mesh=mesh,
        compiler_params=pltpu.CompilerParams(needs_layout_passes=False),
    )
    def sc_kernel(vl_hbm, pr_hbm, tps_hbm, tv_hbm, xpos_hbm, out_hbm,
                  t_v, lbl_v, xpos_v, sems):
        wid = lax.axis_index("s") * nc + lax.axis_index("c")
        base = wid * s_per_w
        pltpu.sync_copy(tps_hbm, t_v.at[0])
        pltpu.sync_copy(tps_hbm, t_v.at[1])
        pltpu.sync_copy(xpos_hbm, xpos_v)
        pltpu.sync_copy(vl_hbm.at[pl.ds(base, s_per_w)], lbl_v)
        xpos_vec = xpos_v[...]
        for j in range(n_ctx):
            p = _lane_extract(xpos_vec, j)
            pltpu.sync_copy(pr_hbm.at[j], t_v.at[0, p])
            pltpu.sync_copy(pr_hbm.at[j], t_v.at[1, p])
        pv = _lane_extract(xpos_vec, n_ctx)
        pltpu.sync_copy(tv_hbm.at[0], t_v.at[0, pv])
        pltpu.sync_copy(tv_hbm.at[1], t_v.at[1, pv])

        def _dma(i, lbl):
            return pltpu.make_async_copy(
                t_v.at[lbl], out_hbm.at[base + i], sems.at[lax.rem(i, NBUF)]
            )

        def chunk_body(c, carry):
            vec = lbl_v[pl.ds(pl.multiple_of(c * LANES, LANES), LANES)]
            for l in range(LANES):
                i = c * LANES + l
                @pl.when(i >= NBUF)
                def _():
                    _dma(i - NBUF, 0).wait()
                lbl = _lane_extract(vec, l)
                _dma(i, lbl).start()
            return carry

        lax.fori_loop(0, n_chunks, chunk_body, 0)
        for k in range(NBUF):
            _dma(s_per_w - NBUF + k, 0).wait()

    return sc_kernel


def kernel(view_label, prompts, token_prefix_suffix, token_view, tokenized_prompts):
    b = view_label.shape[0]
    t, d = token_prefix_suffix.shape[1], token_prefix_suffix.shape[2]
    n_ctx = prompts.shape[1]
    vl = view_label.astype(jnp.int32)
    pr = prompts.reshape(n_ctx, d)
    tps = token_prefix_suffix.reshape(t, d)
    tv = token_view[0, 1:3, :]
    x_pos = jnp.nonzero(tokenized_prompts == X_ID, size=n_ctx + 1)[1]
    xpos = jnp.zeros((LANES,), jnp.int32).at[: n_ctx + 1].set(x_pos.astype(jnp.int32))
    sc = _make_sc_kernel(b, t, d, n_ctx, token_prefix_suffix.dtype)
    return sc(vl, pr, tps, tv, xpos)


# speedup vs baseline: 1.0392x; 1.0392x over previous
import functools
import jax
import jax.numpy as jnp
from jax import lax
from jax.experimental import pallas as pl
from jax.experimental.pallas import tpu as pltpu
from jax.experimental.pallas import tpu_sc as plsc

X_ID = 343
NBUF = 16
LANES = 16


def _lane_extract(vec, lane):
    lanes = lax.iota(jnp.int32, LANES)
    return jnp.max(jnp.where(lanes == lane, vec, -1))


def _make_sc_kernel(b, t, d, n_ctx, dtype):
    info = plsc.get_sparse_core_info()
    nc, ns = info.num_cores, info.num_subcores
    nw = nc * ns
    s_per_w = b // nw
    n_chunks = s_per_w // LANES
    mesh = plsc.VectorSubcoreMesh(core_axis_name="c", subcore_axis_name="s")

    @functools.partial(
        pl.kernel,
        out_type=jax.ShapeDtypeStruct((b, t, d), dtype),
        scratch_types=[
            pltpu.VMEM((2, t, d), dtype),
            pltpu.VMEM((s_per_w,), jnp.int32),
            pltpu.VMEM((LANES,), jnp.int32),
            pltpu.SemaphoreType.DMA((NBUF,)),
        ],
        mesh=mesh,
        compiler_params=pltpu.CompilerParams(needs_layout_passes=False),
    )
    def sc_kernel(vl_hbm, pr_hbm, tps_hbm, tv_hbm, xpos_hbm, out_hbm,
                  t_v, lbl_v, xpos_v, sems):
        wid = lax.axis_index("s") * nc + lax.axis_index("c")
        base = wid * s_per_w
        pltpu.sync_copy(tps_hbm, t_v.at[0])
        pltpu.sync_copy(tps_hbm, t_v.at[1])
        pltpu.sync_copy(xpos_hbm, xpos_v)
        pltpu.sync_copy(vl_hbm.at[pl.ds(base, s_per_w)], lbl_v)
        xpos_vec = xpos_v[...]
        for j in range(n_ctx):
            p = _lane_extract(xpos_vec, j)
            pltpu.sync_copy(pr_hbm.at[j], t_v.at[0, p])
            pltpu.sync_copy(pr_hbm.at[j], t_v.at[1, p])
        pv = _lane_extract(xpos_vec, n_ctx)
        pltpu.sync_copy(tv_hbm.at[0], t_v.at[0, pv])
        pltpu.sync_copy(tv_hbm.at[1], t_v.at[1, pv])

        def _dma(i, lbl):
            return pltpu.make_async_copy(
                t_v.at[lbl], out_hbm.at[base + i], sems.at[lax.rem(i, NBUF)]
            )

        def chunk_body(c, carry):
            vec = lbl_v[pl.ds(pl.multiple_of(c * LANES, LANES), LANES)]
            for l in range(LANES):
                i = c * LANES + l
                @pl.when(i >= NBUF)
                def _():
                    _dma(i - NBUF, 0).wait()
                lbl = _lane_extract(vec, l)
                _dma(i, lbl).start()
            return carry

        lax.fori_loop(0, n_chunks, chunk_body, 0)
        for k in range(NBUF):
            _dma(s_per_w - NBUF + k, 0).wait()

    return sc_kernel


def kernel(view_label, prompts, token_prefix_suffix, token_view, tokenized_prompts):
    b = view_label.shape[0]
    t, d = token_prefix_suffix.shape[1], token_prefix_suffix.shape[2]
    n_ctx = prompts.shape[1]
    vl = view_label.astype(jnp.int32)
    pr = prompts.reshape(n_ctx, d)
    tps = token_prefix_suffix.reshape(t, d)
    tv = token_view[0, 1:3, :]
    x_pos = jnp.nonzero(tokenized_prompts == X_ID, size=n_ctx + 1)[1]
    xpos = jnp.zeros((LANES,), jnp.int32).at[: n_ctx + 1].set(x_pos.astype(jnp.int32))
    sc = _make_sc_kernel(b, t, d, n_ctx, token_prefix_suffix.dtype)
    return sc(vl, pr, tps, tv, xpos)
